# baseline (speedup 1.0000x reference)
import functools

import numpy as np
import jax
import jax.numpy as jnp
from jax.experimental import pallas as pl
from jax.experimental.pallas import tpu as pltpu

DD = 64
KK = 8
SINK_ITERS = 20
NCLS = 41
NVOC = 60
BB, NN, MM, LL = 16, 1024, 32, 12
TOBJ = 512
BM = BB * MM
HI = jax.lax.Precision.HIGHEST

_NORM = float(-np.log(NN + MM))
_LOG_MU_BIN = float(np.log(MM) + _NORM)
_LOG_NU_BIN = float(np.log(NN) + _NORM)


def _dot(a, b, prec=None):
    return jax.lax.dot_general(a, b, (((1,), (0,)), ((), ())), precision=prec)


def _dot_t(a, b, prec=None):
    return jax.lax.dot_general(a, b, (((1,), (1,)), ((), ())), precision=prec)


def _l2n(x):
    return x * jax.lax.rsqrt(jnp.sum(x * x, axis=-1, keepdims=True) + 1e-12)


def _split3(x):
    hi = x.astype(jnp.bfloat16)
    r1 = x - hi.astype(jnp.float32)
    mid = r1.astype(jnp.bfloat16)
    lo = (r1 - mid.astype(jnp.float32)).astype(jnp.bfloat16)
    return hi, mid, lo


def _gather3(selb, parts):
    sel16 = selb.astype(jnp.bfloat16)
    hi, mid, lo = parts
    dims = (((1,), (0,)), ((), ()))
    ghi = jax.lax.dot_general(sel16, hi, dims, preferred_element_type=jnp.float32)
    gmid = jax.lax.dot_general(sel16, mid, dims, preferred_element_type=jnp.float32)
    glo = jax.lax.dot_general(sel16, lo, dims, preferred_element_type=jnp.float32)
    return (ghi + gmid) + glo


def _conv_a(x, W1, n, T, emit):
    ones_row = jnp.ones((1, DD), jnp.float32)
    sq_row = jax.lax.dot_general(ones_row, x * x, (((1,), (1,)), ((), ())),
                                 precision=HI)
    parts = _split3(x)
    s = jnp.zeros((1, DD), jnp.float32)
    ss = jnp.zeros((1, DD), jnp.float32)
    G = n // 128
    for t in range(n // T):
        xt = x[t * T:(t + 1) * T]
        xx = _dot_t(xt, x)
        row_iota = jax.lax.broadcasted_iota(jnp.int32, (T, n), 0) + t * T
        col_iota = jax.lax.broadcasted_iota(jnp.int32, (T, n), 1)
        cur = jnp.where(row_iota == col_iota, jnp.float32(1e30),
                        sq_row - 2.0 * xx)
        for k in range(KK):
            if G >= 2:
                m1 = cur[:, 0:128]
                for g in range(1, G):
                    m1 = jnp.minimum(m1, cur[:, g * 128:(g + 1) * 128])
                m = jnp.min(m1, axis=1, keepdims=True)
                ci128 = jax.lax.broadcasted_iota(jnp.int32, (T, 128), 1)
                idxc = jnp.full((T, 128), n, jnp.int32)
                for g in range(G):
                    idxc = jnp.minimum(
                        idxc, jnp.where(cur[:, g * 128:(g + 1) * 128] == m,
                                        ci128 + g * 128, n))
                idxv = jnp.min(idxc, axis=1, keepdims=True)
            else:
                m = jnp.min(cur, axis=1, keepdims=True)
                idxv = jnp.min(jnp.where(cur == m, col_iota, n),
                               axis=1, keepdims=True)
            selb = col_iota == idxv
            xj = _gather3(selb, parts)
            e = jnp.concatenate([xt, xj - xt], axis=1)
            h = _dot(e, W1)
            emit(k, t, h)
            s = s + jnp.sum(h, axis=0, keepdims=True)
            ss = ss + jnp.sum(h * h, axis=0, keepdims=True)
            cur = jnp.where(selb, jnp.float32(1e30), cur)
    return s, ss


def _conv_b(h_get, s, ss, gam, bet, W2, rows, cnt):
    mu = s * (1.0 / cnt)
    var = ss * (1.0 / cnt) - mu * mu
    sd = jnp.sqrt(var + 1e-5)
    acc = jnp.full((rows, DD), -1e30, jnp.float32)
    for k in range(KK):
        z = jax.nn.relu((h_get(k) - mu) / sd * gam + bet)
        acc = jnp.maximum(acc, _dot(z, W2))
    return acc


def _head(a, bfeat, rW1, rb1, rW2, rb2, pW):
    cat = jnp.concatenate([a, bfeat], axis=1)
    z = _dot(jax.nn.relu(_dot(cat, rW1) + rb1), rW2) + rb2
    return _dot(_l2n(z), pW)


def _k1_kernel(ci_ref, col_ref, pos_ref, cemb_ref,
               cW1_ref, cb1_ref, cW2_ref, cb2_ref,
               pW1_ref, pb1_ref, pW2_ref, pb2_ref,
               mW_ref, mb_ref, W1_ref,
               h_ref, s_ref, ss_ref):
    b = pl.program_id(0)

    @pl.when(b == 0)
    def _init():
        s_ref[...] = jnp.zeros_like(s_ref)
        ss_ref[...] = jnp.zeros_like(ss_ref)

    ci = ci_ref[0]
    oh_t = (ci == jax.lax.broadcasted_iota(jnp.int32, (48, 1), 0)).astype(jnp.float32)
    ce = jax.lax.dot_general(oh_t, cemb_ref[...], (((0,), (0,)), ((), ())),
                             precision=HI)
    ce = _l2n(ce)
    col = _l2n(_dot(jax.nn.relu(_dot(col_ref[0], cW1_ref[...]) + cb1_ref[...]),
                    cW2_ref[...]) + cb2_ref[...])
    pos = _l2n(_dot(jax.nn.relu(_dot(pos_ref[0], pW1_ref[...]) + pb1_ref[...]),
                    pW2_ref[...]) + pb2_ref[...])
    merged = jnp.concatenate([ce, col, pos], axis=1)
    x = _dot(merged, mW_ref[...]) + mb_ref[...]

    def emit(k, t, h):
        h_ref[0, k, t * TOBJ:(t + 1) * TOBJ, :] = h

    s, ss = _conv_a(x, W1_ref[...], NN, TOBJ, emit)
    s_ref[...] += s
    ss_ref[...] += ss


def _k2_kernel(h1_ref, s1_ref, ss1_ref, gam_ref, bet_ref, W2_ref, W1b_ref,
               o1_ref, h2_ref, s2_ref, ss2_ref):
    b = pl.program_id(0)

    @pl.when(b == 0)
    def _init():
        s2_ref[...] = jnp.zeros_like(s2_ref)
        ss2_ref[...] = jnp.zeros_like(ss2_ref)

    o1 = _conv_b(lambda k: h1_ref[0, k], s1_ref[...], ss1_ref[...],
                 gam_ref[...], bet_ref[...], W2_ref[...], NN,
                 float(BB * NN * KK))
    o1_ref[0] = o1

    def emit(k, t, h):
        h2_ref[0, k, t * TOBJ:(t + 1) * TOBJ, :] = h

    s, ss = _conv_a(o1, W1b_ref[...], NN, TOBJ, emit)
    s2_ref[...] += s
    ss2_ref[...] += ss


def _k3_kernel(h2_ref, s2_ref, ss2_ref, gam_ref, bet_ref, W2_ref,
               o1_ref, md1_ref, rW1_ref, rb1_ref, rW2_ref, rb2_ref, pW_ref,
               s3_ref):
    o2 = _conv_b(lambda k: h2_ref[0, k], s2_ref[...], ss2_ref[...],
                 gam_ref[...], bet_ref[...], W2_ref[...], NN,
                 float(BB * NN * KK))
    md0 = _head(o1_ref[0], o2, rW1_ref[...], rb1_ref[...], rW2_ref[...],
                rb2_ref[...], pW_ref[...])
    s3_ref[0] = _dot_t(md0, md1_ref[0]) * (1.0 / np.sqrt(DD))


NH = BB * MM


def _conv_a_stacked(x, W1):
    ones_row = jnp.ones((1, DD), jnp.float32)
    sq_row = jax.lax.dot_general(ones_row, x * x, (((1,), (1,)), ((), ())),
                                 precision=HI)
    parts = _split3(x)
    xx = _dot_t(x, x)
    row_iota = jax.lax.broadcasted_iota(jnp.int32, (NH, NH), 0)
    col_iota = jax.lax.broadcasted_iota(jnp.int32, (NH, NH), 1)
    ok = ((row_iota // MM) == (col_iota // MM)) & (row_iota != col_iota)
    cur = jnp.where(ok, sq_row - 2.0 * xx, jnp.float32(1e30))
    s = jnp.zeros((1, DD), jnp.float32)
    ss = jnp.zeros((1, DD), jnp.float32)
    hs = []
    G = NH // 128
    ci128 = jax.lax.broadcasted_iota(jnp.int32, (NH, 128), 1)
    for k in range(KK):
        m1 = cur[:, 0:128]
        for g in range(1, G):
            m1 = jnp.minimum(m1, cur[:, g * 128:(g + 1) * 128])
        m = jnp.min(m1, axis=1, keepdims=True)
        idxc = jnp.full((NH, 128), NH, jnp.int32)
        for g in range(G):
            idxc = jnp.minimum(
                idxc, jnp.where(cur[:, g * 128:(g + 1) * 128] == m,
                                ci128 + g * 128, NH))
        idxv = jnp.min(idxc, axis=1, keepdims=True)
        selb = col_iota == idxv
        xj = _gather3(selb, parts)
        e = jnp.concatenate([x, xj - x], axis=1)
        h = _dot(e, W1)
        hs.append(h)
        s = s + jnp.sum(h, axis=0, keepdims=True)
        ss = ss + jnp.sum(h * h, axis=0, keepdims=True)
        cur = jnp.where(selb, jnp.float32(1e30), cur)
    return hs, s, ss


def _kh_kernel(tok_ref, wemb_ref, lW_ref, lb_ref,
               g1W1_ref, g1gam_ref, g1bet_ref, g1W2_ref,
               g2W1_ref, g2gam_ref, g2bet_ref, g2W2_ref,
               rW1_ref, rb1_ref, rW2_ref, rb2_ref, pW_ref,
               md1_ref):
    viota = jax.lax.broadcasted_iota(jnp.int32, (1, 64), 1)
    tok = tok_ref[...].reshape(NH, LL)
    counts = jnp.zeros((NH, 64), jnp.float32)
    for l in range(LL):
        counts = counts + (tok[:, l:l + 1] == viota).astype(jnp.float32)
    hint0 = _dot(counts, wemb_ref[...], prec=HI) * (1.0 / LL)
    f = jnp.tanh(_dot(hint0, lW_ref[...]) + lb_ref[...])

    cnt = float(BB * MM * KK)
    hs1, s1, ss1 = _conv_a_stacked(f, g1W1_ref[...])
    o1 = _conv_b(lambda k: hs1[k], s1, ss1, g1gam_ref[...], g1bet_ref[...],
                 g1W2_ref[...], NH, cnt)
    hs2, s2, ss2 = _conv_a_stacked(o1, g2W1_ref[...])
    o2 = _conv_b(lambda k: hs2[k], s2, ss2, g2gam_ref[...], g2bet_ref[...],
                 g2W2_ref[...], NH, cnt)
    md1 = _head(o1, o2, rW1_ref[...], rb1_ref[...], rW2_ref[...],
                rb2_ref[...], pW_ref[...])
    md1_ref[...] = md1.reshape(BB, MM, DD)


def _k4_kernel(s3_ref, a_ref, u_ref, v_ref, ub_ref, vb_ref):
    S = jnp.concatenate([s3_ref[b] for b in range(BB)], axis=1)
    alpha = a_ref[...]
    norm = jnp.float32(_NORM)
    bd = (jax.lax.broadcasted_iota(jnp.int32, (BM, BB), 0) // MM
          == jax.lax.broadcasted_iota(jnp.int32, (BM, BB), 1)
          ).astype(jnp.float32)
    bdt = (jax.lax.broadcasted_iota(jnp.int32, (BB, BM), 0)
           == jax.lax.broadcasted_iota(jnp.int32, (BB, BM), 1) // MM
           ).astype(jnp.float32)

    ES = jnp.exp(S)
    u = jnp.zeros((NN, BB), jnp.float32)
    ub = jnp.zeros((1, BB), jnp.float32)
    v = jnp.zeros((1, BM), jnp.float32)
    vb = jnp.zeros((1, BB), jnp.float32)
    for _ in range(SINK_ITERS):
        ev = jnp.exp(v)
        r = _dot(ES * ev, bd, prec=HI)
        u = norm - jnp.log(r + jnp.exp(alpha + vb))
        rv = _dot(ev, bd, prec=HI)
        ub = jnp.float32(_LOG_MU_BIN) - (alpha + jnp.log(rv + jnp.exp(vb)))
        eu = jnp.exp(u)
        eu_exp = _dot(eu, bdt, prec=HI)
        eub_exp = _dot(jnp.exp(alpha + ub), bdt, prec=HI)
        c = jnp.sum(ES * eu_exp, axis=0, keepdims=True)
        v = norm - jnp.log(c + eub_exp)
        cu = jnp.sum(eu, axis=0, keepdims=True)
        vb = jnp.float32(_LOG_NU_BIN) - (alpha + jnp.log(cu + jnp.exp(ub)))
    for b in range(BB):
        u_ref[b] = u[:, b:b + 1]
        v_ref[b] = v[:, b * MM:(b + 1) * MM]
        ub_ref[b] = ub[:, b:b + 1]
        vb_ref[b] = vb[:, b:b + 1]


def _k5_kernel(s3_ref, u_ref, v_ref, ub_ref, vb_ref, a_ref,
               pmain_ref, prcol_ref, pbrow_ref, pcorn_ref, m0_ref, m1_ref):
    S = s3_ref[0]
    u = u_ref[0]
    v = v_ref[0]
    ub = ub_ref[0]
    vb = vb_ref[0]
    alpha = a_ref[...]
    norm = jnp.float32(_NORM)

    scf = S + u + v - norm
    pmain_ref[0] = jnp.exp(scf)
    prcol_ref[0] = jnp.exp(alpha + u + vb - norm)
    pbrow_ref[0] = jnp.exp(alpha + ub + v - norm)
    pcorn_ref[0] = jnp.exp(alpha + ub + vb - norm)

    iota_i = jax.lax.broadcasted_iota(jnp.int32, (NN, 1), 0)
    iota_j = jax.lax.broadcasted_iota(jnp.int32, (1, MM), 1)
    iota_ij = jax.lax.broadcasted_iota(jnp.int32, (NN, MM), 1)
    iota_ii = jax.lax.broadcasted_iota(jnp.int32, (NN, MM), 0)

    max0 = jnp.max(scf, axis=1, keepdims=True)
    idx0 = jnp.min(jnp.where(scf == max0, iota_ij, MM), axis=1, keepdims=True)
    max1 = jnp.max(scf, axis=0, keepdims=True)
    idx1 = jnp.min(jnp.where(scf == max1, iota_ii, NN), axis=0, keepdims=True)

    oh0 = iota_ij == idx0
    oh1 = iota_ii == idx1
    g0 = jnp.sum(jnp.where(oh0, jnp.broadcast_to(idx1, (NN, MM)), 0),
                 axis=1, keepdims=True)
    mutual0 = iota_i == g0
    g1 = jnp.sum(jnp.where(oh1, jnp.broadcast_to(idx0, (NN, MM)), 0),
                 axis=0, keepdims=True)
    mutual1 = iota_j == g1
    mscores0 = jnp.where(mutual0, jnp.exp(max0), jnp.float32(0.0))
    valid0 = mutual0 & (mscores0 > 0.2)
    gv = jnp.sum(jnp.where(oh1, valid0.astype(jnp.int32), jnp.int32(0)),
                 axis=0, keepdims=True)
    valid1 = mutual1 & (gv > 0)
    m0_ref[0] = jnp.where(valid0, idx0, -1)
    m1_ref[0] = jnp.where(valid1, idx1, -1)


def _w(shape):
    return pl.BlockSpec(shape, lambda b: tuple(0 for _ in shape))


def kernel(class_indices, colors, positions, hint_tokens, class_emb,
           pos_W1, pos_b1, pos_W2, pos_b2, col_W1, col_b1, col_W2, col_b2,
           merge_W, merge_b, g1_W1, g1_gamma, g1_beta, g1_W2,
           g2_W1, g2_gamma, g2_beta, g2_W2, res_W1, res_b1, res_W2, res_b2,
           word_emb, lang_W, lang_b, proj_W, bin_score):
    f32 = jnp.float32
    ci3 = class_indices.astype(jnp.int32).reshape(BB, 1, NN)
    colors_p = jnp.pad(colors, ((0, 0), (0, 0), (0, 5)))
    positions_p = jnp.pad(positions, ((0, 0), (0, 0), (0, 5)))
    cemb_p = jnp.pad(class_emb, ((0, 48 - NCLS), (0, 0)))
    wemb_p = jnp.pad(word_emb, ((0, 64 - NVOC), (0, 0)))
    cW1_p = jnp.pad(col_W1, ((0, 5), (0, 0)))
    pW1_p = jnp.pad(pos_W1, ((0, 5), (0, 0)))
    tok3 = hint_tokens.astype(jnp.int32)
    alpha2 = bin_score.astype(f32).reshape(1, 1)
    g1g = g1_gamma.reshape(1, DD)
    g1b = g1_beta.reshape(1, DD)
    g2g = g2_gamma.reshape(1, DD)
    g2b = g2_beta.reshape(1, DD)
    rb1 = res_b1.reshape(1, DD)
    rb2 = res_b2.reshape(1, DD)

    h1, s1, ss1 = pl.pallas_call(
        _k1_kernel,
        grid=(BB,),
        in_specs=[
            pl.BlockSpec((1, 1, NN), lambda b: (b, 0, 0)),
            pl.BlockSpec((1, NN, 8), lambda b: (b, 0, 0)),
            pl.BlockSpec((1, NN, 8), lambda b: (b, 0, 0)),
            _w((48, DD)), _w((8, 128)), _w((1, 128)), _w((128, DD)),
            _w((1, DD)), _w((8, 128)), _w((1, 128)), _w((128, DD)),
            _w((1, DD)), _w((3 * DD, DD)), _w((1, DD)), _w((2 * DD, DD)),
        ],
        out_specs=[
            pl.BlockSpec((1, KK, NN, DD), lambda b: (b, 0, 0, 0)),
            _w((1, DD)), _w((1, DD)),
        ],
        out_shape=[
            jax.ShapeDtypeStruct((BB, KK, NN, DD), f32),
            jax.ShapeDtypeStruct((1, DD), f32),
            jax.ShapeDtypeStruct((1, DD), f32),
        ],
    )(ci3, colors_p, positions_p, cemb_p,
      cW1_p, col_b1.reshape(1, 128), col_W2, col_b2.reshape(1, DD),
      pW1_p, pos_b1.reshape(1, 128), pos_W2, pos_b2.reshape(1, DD),
      merge_W, merge_b.reshape(1, DD), g1_W1)

    o1, h2, s2, ss2 = pl.pallas_call(
        _k2_kernel,
        grid=(BB,),
        in_specs=[
            pl.BlockSpec((1, KK, NN, DD), lambda b: (b, 0, 0, 0)),
            _w((1, DD)), _w((1, DD)), _w((1, DD)), _w((1, DD)),
            _w((DD, DD)), _w((2 * DD, DD)),
        ],
        out_specs=[
            pl.BlockSpec((1, NN, DD), lambda b: (b, 0, 0)),
            pl.BlockSpec((1, KK, NN, DD), lambda b: (b, 0, 0, 0)),
            _w((1, DD)), _w((1, DD)),
        ],
        out_shape=[
            jax.ShapeDtypeStruct((BB, NN, DD), f32),
            jax.ShapeDtypeStruct((BB, KK, NN, DD), f32),
            jax.ShapeDtypeStruct((1, DD), f32),
            jax.ShapeDtypeStruct((1, DD), f32),
        ],
    )(h1, s1, ss1, g1g, g1b, g1_W2, g2_W1)

    md1 = pl.pallas_call(
        _kh_kernel,
        grid=(1,),
        in_specs=[
            pl.BlockSpec((BB, MM, LL), lambda b: (0, 0, 0)),
            _w((64, DD)), _w((DD, DD)), _w((1, DD)),
            _w((2 * DD, DD)), _w((1, DD)), _w((1, DD)), _w((DD, DD)),
            _w((2 * DD, DD)), _w((1, DD)), _w((1, DD)), _w((DD, DD)),
            _w((2 * DD, DD)), _w((1, DD)), _w((DD, DD)), _w((1, DD)),
            _w((DD, DD)),
        ],
        out_specs=pl.BlockSpec((BB, MM, DD), lambda b: (0, 0, 0)),
        out_shape=jax.ShapeDtypeStruct((BB, MM, DD), f32),
    )(tok3, wemb_p, lang_W, lang_b.reshape(1, DD),
      g1_W1, g1g, g1b, g1_W2, g2_W1, g2g, g2b, g2_W2,
      res_W1, rb1, res_W2, rb2, proj_W)

    s3 = pl.pallas_call(
        _k3_kernel,
        grid=(BB,),
        in_specs=[
            pl.BlockSpec((1, KK, NN, DD), lambda b: (b, 0, 0, 0)),
            _w((1, DD)), _w((1, DD)), _w((1, DD)), _w((1, DD)),
            _w((DD, DD)),
            pl.BlockSpec((1, NN, DD), lambda b: (b, 0, 0)),
            pl.BlockSpec((1, MM, DD), lambda b: (b, 0, 0)),
            _w((2 * DD, DD)), _w((1, DD)), _w((DD, DD)), _w((1, DD)),
            _w((DD, DD)),
        ],
        out_specs=pl.BlockSpec((1, NN, MM), lambda b: (b, 0, 0)),
        out_shape=jax.ShapeDtypeStruct((BB, NN, MM), f32),
    )(h2, s2, ss2, g2g, g2b, g2_W2, o1, md1, res_W1, rb1, res_W2, rb2, proj_W)

    u, v, ubv, vbv = pl.pallas_call(
        _k4_kernel,
        grid=(1,),
        in_specs=[
            pl.BlockSpec((BB, NN, MM), lambda b: (0, 0, 0)),
            _w((1, 1)),
        ],
        out_specs=[
            pl.BlockSpec((BB, NN, 1), lambda b: (0, 0, 0)),
            pl.BlockSpec((BB, 1, MM), lambda b: (0, 0, 0)),
            pl.BlockSpec((BB, 1, 1), lambda b: (0, 0, 0)),
            pl.BlockSpec((BB, 1, 1), lambda b: (0, 0, 0)),
        ],
        out_shape=[
            jax.ShapeDtypeStruct((BB, NN, 1), f32),
            jax.ShapeDtypeStruct((BB, 1, MM), f32),
            jax.ShapeDtypeStruct((BB, 1, 1), f32),
            jax.ShapeDtypeStruct((BB, 1, 1), f32),
        ],
    )(s3, alpha2)

    pmain, prcol, pbrow, pcorn, m0, m1 = pl.pallas_call(
        _k5_kernel,
        grid=(BB,),
        in_specs=[
            pl.BlockSpec((1, NN, MM), lambda b: (b, 0, 0)),
            pl.BlockSpec((1, NN, 1), lambda b: (b, 0, 0)),
            pl.BlockSpec((1, 1, MM), lambda b: (b, 0, 0)),
            pl.BlockSpec((1, 1, 1), lambda b: (b, 0, 0)),
            pl.BlockSpec((1, 1, 1), lambda b: (b, 0, 0)),
            _w((1, 1)),
        ],
        out_specs=[
            pl.BlockSpec((1, NN, MM), lambda b: (b, 0, 0)),
            pl.BlockSpec((1, NN, 1), lambda b: (b, 0, 0)),
            pl.BlockSpec((1, 1, MM), lambda b: (b, 0, 0)),
            pl.BlockSpec((1, 1, 1), lambda b: (b, 0, 0)),
            pl.BlockSpec((1, NN, 1), lambda b: (b, 0, 0)),
            pl.BlockSpec((1, 1, MM), lambda b: (b, 0, 0)),
        ],
        out_shape=[
            jax.ShapeDtypeStruct((BB, NN, MM), f32),
            jax.ShapeDtypeStruct((BB, NN, 1), f32),
            jax.ShapeDtypeStruct((BB, 1, MM), f32),
            jax.ShapeDtypeStruct((BB, 1, 1), f32),
            jax.ShapeDtypeStruct((BB, NN, 1), jnp.int32),
            jax.ShapeDtypeStruct((BB, 1, MM), jnp.int32),
        ],
    )(s3, u, v, ubv, vbv, alpha2)

    P = jnp.concatenate([
        jnp.concatenate([pmain, prcol], axis=2),
        jnp.concatenate([pbrow, pcorn], axis=2),
    ], axis=1)
    matches0 = m0[:, :, 0]
    matches1 = m1[:, 0, :]
    return P, matches0, matches1


# speedup vs baseline: 1.2912x; 1.2912x over previous
import functools

import numpy as np
import jax
import jax.numpy as jnp
from jax.experimental import pallas as pl
from jax.experimental.pallas import tpu as pltpu

DD = 64
KK = 8
SINK_ITERS = 20
NCLS = 41
NVOC = 60
BB, NN, MM, LL = 16, 1024, 32, 12
TOBJ = 256
BM = BB * MM
HI = jax.lax.Precision.HIGHEST

_NORM = float(-np.log(NN + MM))
_LOG_MU_BIN = float(np.log(MM) + _NORM)
_LOG_NU_BIN = float(np.log(NN) + _NORM)


def _dot(a, b, prec=None):
    return jax.lax.dot_general(a, b, (((1,), (0,)), ((), ())), precision=prec)


def _dot_t(a, b, prec=None):
    return jax.lax.dot_general(a, b, (((1,), (1,)), ((), ())), precision=prec)


def _l2n(x):
    return x * jax.lax.rsqrt(jnp.sum(x * x, axis=-1, keepdims=True) + 1e-12)


def _split3(x):
    hi = x.astype(jnp.bfloat16)
    r1 = x - hi.astype(jnp.float32)
    mid = r1.astype(jnp.bfloat16)
    lo = (r1 - mid.astype(jnp.float32)).astype(jnp.bfloat16)
    return hi, mid, lo


def _gather3(selb, parts):
    sel16 = selb.astype(jnp.bfloat16)
    hi, mid, lo = parts
    dims = (((1,), (0,)), ((), ()))
    ghi = jax.lax.dot_general(sel16, hi, dims, preferred_element_type=jnp.float32)
    gmid = jax.lax.dot_general(sel16, mid, dims, preferred_element_type=jnp.float32)
    glo = jax.lax.dot_general(sel16, lo, dims, preferred_element_type=jnp.float32)
    return (ghi + gmid) + glo


def _conv_a(x, W1, n, T, emit):
    ones_row = jnp.ones((1, DD), jnp.float32)
    sq_row = jax.lax.dot_general(ones_row, x * x, (((1,), (1,)), ((), ())),
                                 precision=HI)
    parts = _split3(x)
    s = jnp.zeros((1, DD), jnp.float32)
    ss = jnp.zeros((1, DD), jnp.float32)
    G = n // 128
    for t in range(n // T):
        xt = x[t * T:(t + 1) * T]
        xx = _dot_t(xt, x)
        row_iota = jax.lax.broadcasted_iota(jnp.int32, (T, n), 0) + t * T
        col_iota = jax.lax.broadcasted_iota(jnp.int32, (T, n), 1)
        cur = jnp.where(row_iota == col_iota, jnp.float32(1e30),
                        sq_row - 2.0 * xx)
        for k in range(KK):
            if G >= 2:
                m1 = cur[:, 0:128]
                for g in range(1, G):
                    m1 = jnp.minimum(m1, cur[:, g * 128:(g + 1) * 128])
                m = jnp.min(m1, axis=1, keepdims=True)
                ci128 = jax.lax.broadcasted_iota(jnp.int32, (T, 128), 1)
                idxc = jnp.full((T, 128), n, jnp.int32)
                for g in range(G):
                    idxc = jnp.minimum(
                        idxc, jnp.where(cur[:, g * 128:(g + 1) * 128] == m,
                                        ci128 + g * 128, n))
                idxv = jnp.min(idxc, axis=1, keepdims=True)
            else:
                m = jnp.min(cur, axis=1, keepdims=True)
                idxv = jnp.min(jnp.where(cur == m, col_iota, n),
                               axis=1, keepdims=True)
            selb = col_iota == idxv
            xj = _gather3(selb, parts)
            e = jnp.concatenate([xt, xj - xt], axis=1)
            h = _dot(e, W1)
            emit(k, t, h)
            s = s + jnp.sum(h, axis=0, keepdims=True)
            ss = ss + jnp.sum(h * h, axis=0, keepdims=True)
            cur = jnp.where(selb, jnp.float32(1e30), cur)
    return s, ss


def _conv_b(h_get, s, ss, gam, bet, W2, rows, cnt):
    mu = s * (1.0 / cnt)
    var = ss * (1.0 / cnt) - mu * mu
    sd = jnp.sqrt(var + 1e-5)
    acc = jnp.full((rows, DD), -1e30, jnp.float32)
    for k in range(KK):
        z = jax.nn.relu((h_get(k) - mu) / sd * gam + bet)
        acc = jnp.maximum(acc, _dot(z, W2))
    return acc


def _head(a, bfeat, rW1, rb1, rW2, rb2, pW):
    cat = jnp.concatenate([a, bfeat], axis=1)
    z = _dot(jax.nn.relu(_dot(cat, rW1) + rb1), rW2) + rb2
    return _dot(_l2n(z), pW)


def _k1_kernel(ci_ref, col_ref, pos_ref, cemb_ref,
               cW1_ref, cb1_ref, cW2_ref, cb2_ref,
               pW1_ref, pb1_ref, pW2_ref, pb2_ref,
               mW_ref, mb_ref, W1_ref,
               h_ref, s_ref, ss_ref):
    b = pl.program_id(0)

    @pl.when(b == 0)
    def _init():
        s_ref[...] = jnp.zeros_like(s_ref)
        ss_ref[...] = jnp.zeros_like(ss_ref)

    ci = ci_ref[0]
    oh_t = (ci == jax.lax.broadcasted_iota(jnp.int32, (48, 1), 0)).astype(jnp.float32)
    ce = jax.lax.dot_general(oh_t, cemb_ref[...], (((0,), (0,)), ((), ())),
                             precision=HI)
    ce = _l2n(ce)
    col = _l2n(_dot(jax.nn.relu(_dot(col_ref[0], cW1_ref[...]) + cb1_ref[...]),
                    cW2_ref[...]) + cb2_ref[...])
    pos = _l2n(_dot(jax.nn.relu(_dot(pos_ref[0], pW1_ref[...]) + pb1_ref[...]),
                    pW2_ref[...]) + pb2_ref[...])
    merged = jnp.concatenate([ce, col, pos], axis=1)
    x = _dot(merged, mW_ref[...]) + mb_ref[...]

    def emit(k, t, h):
        h_ref[0, k, t * TOBJ:(t + 1) * TOBJ, :] = h

    s, ss = _conv_a(x, W1_ref[...], NN, TOBJ, emit)
    s_ref[...] += s
    ss_ref[...] += ss


def _k2_kernel(h1_ref, s1_ref, ss1_ref, gam_ref, bet_ref, W2_ref, W1b_ref,
               o1_ref, h2_ref, s2_ref, ss2_ref):
    b = pl.program_id(0)

    @pl.when(b == 0)
    def _init():
        s2_ref[...] = jnp.zeros_like(s2_ref)
        ss2_ref[...] = jnp.zeros_like(ss2_ref)

    o1 = _conv_b(lambda k: h1_ref[0, k], s1_ref[...], ss1_ref[...],
                 gam_ref[...], bet_ref[...], W2_ref[...], NN,
                 float(BB * NN * KK))
    o1_ref[0] = o1

    def emit(k, t, h):
        h2_ref[0, k, t * TOBJ:(t + 1) * TOBJ, :] = h

    s, ss = _conv_a(o1, W1b_ref[...], NN, TOBJ, emit)
    s2_ref[...] += s
    ss2_ref[...] += ss


def _k3_kernel(h2_ref, s2_ref, ss2_ref, gam_ref, bet_ref, W2_ref,
               o1_ref, md1_ref, rW1_ref, rb1_ref, rW2_ref, rb2_ref, pW_ref,
               s3_ref):
    o2 = _conv_b(lambda k: h2_ref[0, k], s2_ref[...], ss2_ref[...],
                 gam_ref[...], bet_ref[...], W2_ref[...], NN,
                 float(BB * NN * KK))
    md0 = _head(o1_ref[0], o2, rW1_ref[...], rb1_ref[...], rW2_ref[...],
                rb2_ref[...], pW_ref[...])
    s3_ref[0] = _dot_t(md0, md1_ref[0]) * (1.0 / np.sqrt(DD))


NH = BB * MM


def _conv_a_stacked(x, W1):
    ones_row = jnp.ones((1, DD), jnp.float32)
    sq_row = jax.lax.dot_general(ones_row, x * x, (((1,), (1,)), ((), ())),
                                 precision=HI)
    parts = _split3(x)
    xx = _dot_t(x, x)
    row_iota = jax.lax.broadcasted_iota(jnp.int32, (NH, NH), 0)
    col_iota = jax.lax.broadcasted_iota(jnp.int32, (NH, NH), 1)
    ok = ((row_iota // MM) == (col_iota // MM)) & (row_iota != col_iota)
    cur = jnp.where(ok, sq_row - 2.0 * xx, jnp.float32(1e30))
    s = jnp.zeros((1, DD), jnp.float32)
    ss = jnp.zeros((1, DD), jnp.float32)
    hs = []
    G = NH // 128
    ci128 = jax.lax.broadcasted_iota(jnp.int32, (NH, 128), 1)
    for k in range(KK):
        m1 = cur[:, 0:128]
        for g in range(1, G):
            m1 = jnp.minimum(m1, cur[:, g * 128:(g + 1) * 128])
        m = jnp.min(m1, axis=1, keepdims=True)
        idxc = jnp.full((NH, 128), NH, jnp.int32)
        for g in range(G):
            idxc = jnp.minimum(
                idxc, jnp.where(cur[:, g * 128:(g + 1) * 128] == m,
                                ci128 + g * 128, NH))
        idxv = jnp.min(idxc, axis=1, keepdims=True)
        selb = col_iota == idxv
        xj = _gather3(selb, parts)
        e = jnp.concatenate([x, xj - x], axis=1)
        h = _dot(e, W1)
        hs.append(h)
        s = s + jnp.sum(h, axis=0, keepdims=True)
        ss = ss + jnp.sum(h * h, axis=0, keepdims=True)
        cur = jnp.where(selb, jnp.float32(1e30), cur)
    return hs, s, ss


def _kh_kernel(tok_ref, wemb_ref, lW_ref, lb_ref,
               g1W1_ref, g1gam_ref, g1bet_ref, g1W2_ref,
               g2W1_ref, g2gam_ref, g2bet_ref, g2W2_ref,
               rW1_ref, rb1_ref, rW2_ref, rb2_ref, pW_ref,
               md1_ref):
    viota = jax.lax.broadcasted_iota(jnp.int32, (1, 64), 1)
    tok = tok_ref[...].reshape(NH, LL)
    counts = jnp.zeros((NH, 64), jnp.float32)
    for l in range(LL):
        counts = counts + (tok[:, l:l + 1] == viota).astype(jnp.float32)
    hint0 = _dot(counts, wemb_ref[...], prec=HI) * (1.0 / LL)
    f = jnp.tanh(_dot(hint0, lW_ref[...]) + lb_ref[...])

    cnt = float(BB * MM * KK)
    hs1, s1, ss1 = _conv_a_stacked(f, g1W1_ref[...])
    o1 = _conv_b(lambda k: hs1[k], s1, ss1, g1gam_ref[...], g1bet_ref[...],
                 g1W2_ref[...], NH, cnt)
    hs2, s2, ss2 = _conv_a_stacked(o1, g2W1_ref[...])
    o2 = _conv_b(lambda k: hs2[k], s2, ss2, g2gam_ref[...], g2bet_ref[...],
                 g2W2_ref[...], NH, cnt)
    md1 = _head(o1, o2, rW1_ref[...], rb1_ref[...], rW2_ref[...],
                rb2_ref[...], pW_ref[...])
    md1_ref[...] = md1.reshape(BB, MM, DD)


def _k4_kernel(s3_ref, a_ref, u_ref, v_ref, ub_ref, vb_ref):
    S = jnp.concatenate([s3_ref[b] for b in range(BB)], axis=1)
    alpha = a_ref[...]
    norm = jnp.float32(_NORM)
    bd = (jax.lax.broadcasted_iota(jnp.int32, (BM, BB), 0) // MM
          == jax.lax.broadcasted_iota(jnp.int32, (BM, BB), 1)
          ).astype(jnp.float32)
    bdt = (jax.lax.broadcasted_iota(jnp.int32, (BB, BM), 0)
           == jax.lax.broadcasted_iota(jnp.int32, (BB, BM), 1) // MM
           ).astype(jnp.float32)

    bd16 = bd.astype(jnp.bfloat16)
    bdt16 = bdt.astype(jnp.bfloat16)
    dims = (((1,), (0,)), ((), ()))

    def seg(x, m16):
        hi, mid, lo = _split3(x)
        ghi = jax.lax.dot_general(hi, m16, dims, preferred_element_type=jnp.float32)
        gmid = jax.lax.dot_general(mid, m16, dims, preferred_element_type=jnp.float32)
        glo = jax.lax.dot_general(lo, m16, dims, preferred_element_type=jnp.float32)
        return (ghi + gmid) + glo

    ES = jnp.exp(S)
    u = jnp.zeros((NN, BB), jnp.float32)
    ub = jnp.zeros((1, BB), jnp.float32)
    v = jnp.zeros((1, BM), jnp.float32)
    vb = jnp.zeros((1, BB), jnp.float32)
    for _ in range(SINK_ITERS):
        ev = jnp.exp(v)
        r = seg(ES * ev, bd16)
        u = norm - jnp.log(r + jnp.exp(alpha + vb))
        rv = seg(ev, bd16)
        ub = jnp.float32(_LOG_MU_BIN) - (alpha + jnp.log(rv + jnp.exp(vb)))
        eu = jnp.exp(u)
        eu_exp = seg(eu, bdt16)
        eub_exp = seg(jnp.exp(alpha + ub), bdt16)
        c = jnp.sum(ES * eu_exp, axis=0, keepdims=True)
        v = norm - jnp.log(c + eub_exp)
        cu = jnp.sum(eu, axis=0, keepdims=True)
        vb = jnp.float32(_LOG_NU_BIN) - (alpha + jnp.log(cu + jnp.exp(ub)))
    for b in range(BB):
        u_ref[b] = u[:, b:b + 1]
        v_ref[b] = v[:, b * MM:(b + 1) * MM]
        ub_ref[b] = ub[:, b:b + 1]
        vb_ref[b] = vb[:, b:b + 1]


def _k5_kernel(s3_ref, u_ref, v_ref, ub_ref, vb_ref, a_ref,
               pmain_ref, prcol_ref, pbrow_ref, pcorn_ref, m0_ref, m1_ref):
    S = s3_ref[0]
    u = u_ref[0]
    v = v_ref[0]
    ub = ub_ref[0]
    vb = vb_ref[0]
    alpha = a_ref[...]
    norm = jnp.float32(_NORM)

    scf = S + u + v - norm
    pmain_ref[0] = jnp.exp(scf)
    prcol_ref[0] = jnp.exp(alpha + u + vb - norm)
    pbrow_ref[0] = jnp.exp(alpha + ub + v - norm)
    pcorn_ref[0] = jnp.exp(alpha + ub + vb - norm)

    iota_i = jax.lax.broadcasted_iota(jnp.int32, (NN, 1), 0)
    iota_j = jax.lax.broadcasted_iota(jnp.int32, (1, MM), 1)
    iota_ij = jax.lax.broadcasted_iota(jnp.int32, (NN, MM), 1)
    iota_ii = jax.lax.broadcasted_iota(jnp.int32, (NN, MM), 0)

    max0 = jnp.max(scf, axis=1, keepdims=True)
    idx0 = jnp.min(jnp.where(scf == max0, iota_ij, MM), axis=1, keepdims=True)
    max1 = jnp.max(scf, axis=0, keepdims=True)
    idx1 = jnp.min(jnp.where(scf == max1, iota_ii, NN), axis=0, keepdims=True)

    oh0 = iota_ij == idx0
    oh1 = iota_ii == idx1
    g0 = jnp.sum(jnp.where(oh0, jnp.broadcast_to(idx1, (NN, MM)), 0),
                 axis=1, keepdims=True)
    mutual0 = iota_i == g0
    g1 = jnp.sum(jnp.where(oh1, jnp.broadcast_to(idx0, (NN, MM)), 0),
                 axis=0, keepdims=True)
    mutual1 = iota_j == g1
    mscores0 = jnp.where(mutual0, jnp.exp(max0), jnp.float32(0.0))
    valid0 = mutual0 & (mscores0 > 0.2)
    gv = jnp.sum(jnp.where(oh1, valid0.astype(jnp.int32), jnp.int32(0)),
                 axis=0, keepdims=True)
    valid1 = mutual1 & (gv > 0)
    m0_ref[0] = jnp.where(valid0, idx0, -1)
    m1_ref[0] = jnp.where(valid1, idx1, -1)


def _w(shape):
    return pl.BlockSpec(shape, lambda b: tuple(0 for _ in shape))


def kernel(class_indices, colors, positions, hint_tokens, class_emb,
           pos_W1, pos_b1, pos_W2, pos_b2, col_W1, col_b1, col_W2, col_b2,
           merge_W, merge_b, g1_W1, g1_gamma, g1_beta, g1_W2,
           g2_W1, g2_gamma, g2_beta, g2_W2, res_W1, res_b1, res_W2, res_b2,
           word_emb, lang_W, lang_b, proj_W, bin_score):
    f32 = jnp.float32
    ci3 = class_indices.astype(jnp.int32).reshape(BB, 1, NN)
    colors_p = jnp.pad(colors, ((0, 0), (0, 0), (0, 5)))
    positions_p = jnp.pad(positions, ((0, 0), (0, 0), (0, 5)))
    cemb_p = jnp.pad(class_emb, ((0, 48 - NCLS), (0, 0)))
    wemb_p = jnp.pad(word_emb, ((0, 64 - NVOC), (0, 0)))
    cW1_p = jnp.pad(col_W1, ((0, 5), (0, 0)))
    pW1_p = jnp.pad(pos_W1, ((0, 5), (0, 0)))
    tok3 = hint_tokens.astype(jnp.int32)
    alpha2 = bin_score.astype(f32).reshape(1, 1)
    g1g = g1_gamma.reshape(1, DD)
    g1b = g1_beta.reshape(1, DD)
    g2g = g2_gamma.reshape(1, DD)
    g2b = g2_beta.reshape(1, DD)
    rb1 = res_b1.reshape(1, DD)
    rb2 = res_b2.reshape(1, DD)

    h1, s1, ss1 = pl.pallas_call(
        _k1_kernel,
        grid=(BB,),
        in_specs=[
            pl.BlockSpec((1, 1, NN), lambda b: (b, 0, 0)),
            pl.BlockSpec((1, NN, 8), lambda b: (b, 0, 0)),
            pl.BlockSpec((1, NN, 8), lambda b: (b, 0, 0)),
            _w((48, DD)), _w((8, 128)), _w((1, 128)), _w((128, DD)),
            _w((1, DD)), _w((8, 128)), _w((1, 128)), _w((128, DD)),
            _w((1, DD)), _w((3 * DD, DD)), _w((1, DD)), _w((2 * DD, DD)),
        ],
        out_specs=[
            pl.BlockSpec((1, KK, NN, DD), lambda b: (b, 0, 0, 0)),
            _w((1, DD)), _w((1, DD)),
        ],
        out_shape=[
            jax.ShapeDtypeStruct((BB, KK, NN, DD), f32),
            jax.ShapeDtypeStruct((1, DD), f32),
            jax.ShapeDtypeStruct((1, DD), f32),
        ],
    )(ci3, colors_p, positions_p, cemb_p,
      cW1_p, col_b1.reshape(1, 128), col_W2, col_b2.reshape(1, DD),
      pW1_p, pos_b1.reshape(1, 128), pos_W2, pos_b2.reshape(1, DD),
      merge_W, merge_b.reshape(1, DD), g1_W1)

    o1, h2, s2, ss2 = pl.pallas_call(
        _k2_kernel,
        grid=(BB,),
        in_specs=[
            pl.BlockSpec((1, KK, NN, DD), lambda b: (b, 0, 0, 0)),
            _w((1, DD)), _w((1, DD)), _w((1, DD)), _w((1, DD)),
            _w((DD, DD)), _w((2 * DD, DD)),
        ],
        out_specs=[
            pl.BlockSpec((1, NN, DD), lambda b: (b, 0, 0)),
            pl.BlockSpec((1, KK, NN, DD), lambda b: (b, 0, 0, 0)),
            _w((1, DD)), _w((1, DD)),
        ],
        out_shape=[
            jax.ShapeDtypeStruct((BB, NN, DD), f32),
            jax.ShapeDtypeStruct((BB, KK, NN, DD), f32),
            jax.ShapeDtypeStruct((1, DD), f32),
            jax.ShapeDtypeStruct((1, DD), f32),
        ],
    )(h1, s1, ss1, g1g, g1b, g1_W2, g2_W1)

    md1 = pl.pallas_call(
        _kh_kernel,
        grid=(1,),
        in_specs=[
            pl.BlockSpec((BB, MM, LL), lambda b: (0, 0, 0)),
            _w((64, DD)), _w((DD, DD)), _w((1, DD)),
            _w((2 * DD, DD)), _w((1, DD)), _w((1, DD)), _w((DD, DD)),
            _w((2 * DD, DD)), _w((1, DD)), _w((1, DD)), _w((DD, DD)),
            _w((2 * DD, DD)), _w((1, DD)), _w((DD, DD)), _w((1, DD)),
            _w((DD, DD)),
        ],
        out_specs=pl.BlockSpec((BB, MM, DD), lambda b: (0, 0, 0)),
        out_shape=jax.ShapeDtypeStruct((BB, MM, DD), f32),
    )(tok3, wemb_p, lang_W, lang_b.reshape(1, DD),
      g1_W1, g1g, g1b, g1_W2, g2_W1, g2g, g2b, g2_W2,
      res_W1, rb1, res_W2, rb2, proj_W)

    s3 = pl.pallas_call(
        _k3_kernel,
        grid=(BB,),
        in_specs=[
            pl.BlockSpec((1, KK, NN, DD), lambda b: (b, 0, 0, 0)),
            _w((1, DD)), _w((1, DD)), _w((1, DD)), _w((1, DD)),
            _w((DD, DD)),
            pl.BlockSpec((1, NN, DD), lambda b: (b, 0, 0)),
            pl.BlockSpec((1, MM, DD), lambda b: (b, 0, 0)),
            _w((2 * DD, DD)), _w((1, DD)), _w((DD, DD)), _w((1, DD)),
            _w((DD, DD)),
        ],
        out_specs=pl.BlockSpec((1, NN, MM), lambda b: (b, 0, 0)),
        out_shape=jax.ShapeDtypeStruct((BB, NN, MM), f32),
    )(h2, s2, ss2, g2g, g2b, g2_W2, o1, md1, res_W1, rb1, res_W2, rb2, proj_W)

    u, v, ubv, vbv = pl.pallas_call(
        _k4_kernel,
        grid=(1,),
        in_specs=[
            pl.BlockSpec((BB, NN, MM), lambda b: (0, 0, 0)),
            _w((1, 1)),
        ],
        out_specs=[
            pl.BlockSpec((BB, NN, 1), lambda b: (0, 0, 0)),
            pl.BlockSpec((BB, 1, MM), lambda b: (0, 0, 0)),
            pl.BlockSpec((BB, 1, 1), lambda b: (0, 0, 0)),
            pl.BlockSpec((BB, 1, 1), lambda b: (0, 0, 0)),
        ],
        out_shape=[
            jax.ShapeDtypeStruct((BB, NN, 1), f32),
            jax.ShapeDtypeStruct((BB, 1, MM), f32),
            jax.ShapeDtypeStruct((BB, 1, 1), f32),
            jax.ShapeDtypeStruct((BB, 1, 1), f32),
        ],
    )(s3, alpha2)

    pmain, prcol, pbrow, pcorn, m0, m1 = pl.pallas_call(
        _k5_kernel,
        grid=(BB,),
        in_specs=[
            pl.BlockSpec((1, NN, MM), lambda b: (b, 0, 0)),
            pl.BlockSpec((1, NN, 1), lambda b: (b, 0, 0)),
            pl.BlockSpec((1, 1, MM), lambda b: (b, 0, 0)),
            pl.BlockSpec((1, 1, 1), lambda b: (b, 0, 0)),
            pl.BlockSpec((1, 1, 1), lambda b: (b, 0, 0)),
            _w((1, 1)),
        ],
        out_specs=[
            pl.BlockSpec((1, NN, MM), lambda b: (b, 0, 0)),
            pl.BlockSpec((1, NN, 1), lambda b: (b, 0, 0)),
            pl.BlockSpec((1, 1, MM), lambda b: (b, 0, 0)),
            pl.BlockSpec((1, 1, 1), lambda b: (b, 0, 0)),
            pl.BlockSpec((1, NN, 1), lambda b: (b, 0, 0)),
            pl.BlockSpec((1, 1, MM), lambda b: (b, 0, 0)),
        ],
        out_shape=[
            jax.ShapeDtypeStruct((BB, NN, MM), f32),
            jax.ShapeDtypeStruct((BB, NN, 1), f32),
            jax.ShapeDtypeStruct((BB, 1, MM), f32),
            jax.ShapeDtypeStruct((BB, 1, 1), f32),
            jax.ShapeDtypeStruct((BB, NN, 1), jnp.int32),
            jax.ShapeDtypeStruct((BB, 1, MM), jnp.int32),
        ],
    )(s3, u, v, ubv, vbv, alpha2)

    P = jnp.concatenate([
        jnp.concatenate([pmain, prcol], axis=2),
        jnp.concatenate([pbrow, pcorn], axis=2),
    ], axis=1)
    matches0 = m0[:, :, 0]
    matches1 = m1[:, 0, :]
    return P, matches0, matches1
